# baseline (speedup 1.0000x reference)
import functools

import jax
import jax.numpy as jnp
from jax import lax
from jax.experimental import pallas as pl
from jax.experimental.pallas import tpu as pltpu
from jax.experimental.pallas import tpu_sc as plsc

N_NODES = 10000
N_EDGES = 320000
D_IN = 128
D_HID = 128
D_OUT = 2
D_OUT_PAD = 16
N_GRAPHS = 64

NC = 2
NS = 16
NW = NC * NS

NP = 10240
EP = 327680
CH = 64
NCHK = EP // (NW * CH)
RPT = NP // NS
RB = RPT // CH
NCHK_A = 80
NCHK_B = 2 * NCHK - NCHK_A


def _mesh():
    return plsc.VectorSubcoreMesh(core_axis_name="c", subcore_axis_name="s")


def _zero_rows(buf, d):
    def fill(r, _):
        for k in range(d // 16):
            buf[r, pl.ds(k * 16, 16)] = jnp.zeros((16,), jnp.float32)
        return 0
    lax.fori_loop(0, CH, fill, 0)


def _deg_kernel(dst2d):
    @functools.partial(
        pl.kernel,
        mesh=_mesh(),
        compiler_params=pltpu.CompilerParams(use_tc_tiling_on_sc=False),
        out_type=jax.ShapeDtypeStruct((NC, NP, 16), jnp.float32),
        scratch_types=[
            pltpu.VMEM((NCHK, CH), jnp.int32),
            pltpu.VMEM((CH, 16), jnp.float32),
            pltpu.VMEM_SHARED((NP, 16), jnp.float32),
        ],
    )
    def k(dst_hbm, out_hbm, didx, ones, acc_sh):
        c = lax.axis_index("c")
        s = lax.axis_index("s")
        wid = c * NS + s
        _zero_rows(ones, 16)
        for b in range(RB):
            pltpu.sync_copy(ones, acc_sh.at[pl.ds(s * RPT + b * CH, CH)])
        def fill(r, _):
            ones[r, pl.ds(0, 16)] = jnp.ones((16,), jnp.float32)
            return 0
        lax.fori_loop(0, CH, fill, 0)
        pltpu.sync_copy(dst_hbm.at[pl.ds(wid * NCHK, NCHK)], didx)
        plsc.subcore_barrier()
        def step(j, _):
            pltpu.sync_copy(ones, acc_sh.at[didx.at[j]], add=True)
            return 0
        lax.fori_loop(0, NCHK, step, 0)
        plsc.subcore_barrier()
        for b in range(RB):
            r0 = s * RPT + b * CH
            pltpu.sync_copy(acc_sh.at[pl.ds(r0, CH)], out_hbm.at[c, pl.ds(r0, CH)])

    return k(dst2d)


def _spmm_kernel(src2d, dst2d, g, d):
    @functools.partial(
        pl.kernel,
        mesh=_mesh(),
        compiler_params=pltpu.CompilerParams(use_tc_tiling_on_sc=False),
        out_type=jax.ShapeDtypeStruct((NC, NP, d), jnp.float32),
        scratch_types=[
            pltpu.VMEM((NCHK_B, CH), jnp.int32),
            pltpu.VMEM((NCHK_B, CH), jnp.int32),
            pltpu.VMEM((CH, d), jnp.float32),
            pltpu.VMEM((CH, d), jnp.float32),
            pltpu.VMEM_SHARED((NP, d), jnp.float32),
            pltpu.SemaphoreType.DMA,
            pltpu.SemaphoreType.DMA,
            pltpu.SemaphoreType.DMA,
            pltpu.SemaphoreType.DMA,
        ],
    )
    def k(src_hbm, dst_hbm, g_hbm, out_hbm, sidx, didx, r0, r1, acc_sh,
          g0, g1, s0, s1):
        rows = [r0, r1]
        gsem = [g0, g1]
        ssem = [s0, s1]
        c = lax.axis_index("c")
        s = lax.axis_index("s")

        def fire_gather(j, slot):
            pltpu.async_copy(g_hbm.at[sidx.at[j]], rows[slot], gsem[slot])

        def wait_gather(slot):
            pltpu.make_async_copy(g_hbm.at[sidx.at[0]], rows[slot],
                                  gsem[slot]).wait()

        def fire_scatter(j, slot):
            pltpu.async_copy(rows[slot], acc_sh.at[didx.at[j]], ssem[slot],
                             add=True)

        def wait_scatter(slot):
            pltpu.make_async_copy(rows[slot], acc_sh.at[didx.at[0]],
                                  ssem[slot]).wait()

        _zero_rows(r0, d)
        for b in range(RB):
            pltpu.sync_copy(r0, acc_sh.at[pl.ds(s * RPT + b * CH, CH)])

        def run(nchk, base):
            pltpu.sync_copy(src_hbm.at[pl.ds(base, nchk)],
                            sidx.at[pl.ds(0, nchk)])
            pltpu.sync_copy(dst_hbm.at[pl.ds(base, nchk)],
                            didx.at[pl.ds(0, nchk)])
            fire_gather(0, 0)
            wait_gather(0)
            fire_gather(1, 1)
            fire_scatter(0, 0)

            def step(i, _):
                j = 2 * i + 1
                wait_gather(1)
                wait_scatter(0)
                fire_gather(j + 1, 0)
                fire_scatter(j, 1)
                wait_gather(0)
                wait_scatter(1)
                fire_gather(j + 2, 1)
                fire_scatter(j + 1, 0)
                return 0
            lax.fori_loop(0, nchk // 2 - 1, step, 0)
            wait_gather(1)
            fire_scatter(nchk - 1, 1)
            wait_scatter(0)
            wait_scatter(1)

        @pl.when(c == 0)
        def _():
            run(NCHK_A, s * NCHK_A)

        @pl.when(c == 1)
        def _():
            run(NCHK_B, NS * NCHK_A + s * NCHK_B)
        plsc.subcore_barrier()
        for b in range(RB):
            r0 = s * RPT + b * CH
            pltpu.sync_copy(acc_sh.at[pl.ds(r0, CH)], out_hbm.at[c, pl.ds(r0, CH)])

    return k(src2d, dst2d, g)


def _prep1_body(x_ref, w_ref, deg_ref, g_ref):
    hist = deg_ref[0, :, 0:1] + deg_ref[1, :, 0:1]
    dis = lax.rsqrt(hist + 1.0)
    h = jnp.dot(x_ref[...], w_ref[...], preferred_element_type=jnp.float32)
    g_ref[...] = h * dis


def _prep2_body(acc_ref, g1_ref, deg_ref, w_ref, b1_ref, g2_ref):
    hist = deg_ref[0, :, 0:1] + deg_ref[1, :, 0:1]
    dis = lax.rsqrt(hist + 1.0)
    z = dis * (acc_ref[0] + acc_ref[1] + g1_ref[...]) + b1_ref[...]
    z = jnp.maximum(z, 0.0)
    h2 = jnp.dot(z, w_ref[...], preferred_element_type=jnp.float32)
    g2_ref[...] = h2 * dis


def _final_body(acc_ref, g2_ref, deg_ref, b2_ref, batch_ref, out_ref):
    hist = deg_ref[0, :, 0:1] + deg_ref[1, :, 0:1]
    dis = lax.rsqrt(hist + 1.0)
    out2 = dis * (acc_ref[0] + acc_ref[1] + g2_ref[...]) + b2_ref[...]
    ohT = (lax.broadcasted_iota(jnp.int32, (N_GRAPHS, NP), 0)
           == batch_ref[...]).astype(jnp.float32)
    sums = jnp.dot(ohT, out2, preferred_element_type=jnp.float32)
    counts = jnp.sum(ohT, axis=1, keepdims=True)
    pooled = sums / jnp.maximum(counts, 1.0)
    colmask = lax.broadcasted_iota(jnp.int32, (N_GRAPHS, D_OUT_PAD), 1) < D_OUT
    xm = jnp.where(colmask, pooled, -1e30)
    m = jnp.max(xm, axis=1, keepdims=True)
    e = jnp.where(colmask, jnp.exp(xm - m), 0.0)
    lse = jnp.log(jnp.sum(e, axis=1, keepdims=True)) + m
    out_ref[...] = pooled - lse


def kernel(x, edge_index, batch, W1, b1, W2, b2):
    f32 = jnp.float32
    src = jnp.concatenate([edge_index[0], jnp.full((EP - N_EDGES,), N_NODES, jnp.int32)])
    dst = jnp.concatenate([edge_index[1], jnp.full((EP - N_EDGES,), N_NODES, jnp.int32)])
    src2d = src.reshape(EP // CH, CH)
    dst2d = dst.reshape(EP // CH, CH)
    xp = jnp.zeros((NP, D_IN), f32).at[:N_NODES].set(x)
    W2p = jnp.zeros((D_HID, D_OUT_PAD), f32).at[:, :D_OUT].set(W2)
    b1r = b1.reshape(1, D_HID)
    b2r = jnp.zeros((1, D_OUT_PAD), f32).at[0, :D_OUT].set(b2)
    batch2d = jnp.full((1, NP), N_GRAPHS, jnp.int32).at[0, :N_NODES].set(batch)

    degp = _deg_kernel(dst2d)

    nblk, rows = 8, NP // 8
    g1 = pl.pallas_call(
        _prep1_body,
        grid=(nblk,),
        in_specs=[
            pl.BlockSpec((rows, D_IN), lambda i: (i, 0)),
            pl.BlockSpec((D_IN, D_HID), lambda i: (0, 0)),
            pl.BlockSpec((NC, rows, 16), lambda i: (0, i, 0)),
        ],
        out_specs=pl.BlockSpec((rows, D_HID), lambda i: (i, 0)),
        out_shape=jax.ShapeDtypeStruct((NP, D_HID), f32),
    )(xp, W1, degp)

    acc1 = _spmm_kernel(src2d, dst2d, g1, D_HID)

    g2 = pl.pallas_call(
        _prep2_body,
        grid=(nblk,),
        in_specs=[
            pl.BlockSpec((NC, rows, D_HID), lambda i: (0, i, 0)),
            pl.BlockSpec((rows, D_HID), lambda i: (i, 0)),
            pl.BlockSpec((NC, rows, 16), lambda i: (0, i, 0)),
            pl.BlockSpec((D_HID, D_OUT_PAD), lambda i: (0, 0)),
            pl.BlockSpec((1, D_HID), lambda i: (0, 0)),
        ],
        out_specs=pl.BlockSpec((rows, D_OUT_PAD), lambda i: (i, 0)),
        out_shape=jax.ShapeDtypeStruct((NP, D_OUT_PAD), f32),
    )(acc1, g1, degp, W2p, b1r)

    acc2 = _spmm_kernel(src2d, dst2d, g2, D_OUT_PAD)

    res = pl.pallas_call(
        _final_body,
        out_shape=jax.ShapeDtypeStruct((N_GRAPHS, D_OUT_PAD), f32),
    )(acc2, g2, degp, b2r, batch2d)
    return res[:, :D_OUT]


# speedup vs baseline: 2.1235x; 2.1235x over previous
import functools

import jax
import jax.numpy as jnp
from jax import lax
from jax.experimental import pallas as pl
from jax.experimental.pallas import tpu as pltpu
from jax.experimental.pallas import tpu_sc as plsc

N_NODES = 10000
N_EDGES = 320000
D_IN = 128
D_HID = 128
D_OUT = 2
D_OUT_PAD = 16
N_GRAPHS = 64

NC = 2
NS = 16
NW = NC * NS

NP = 10240
EP = 327680
CH = 64
NCHK = EP // (NW * CH)
RPT = NP // NS
RB = RPT // CH
NCHK_A = 160
NCHK_B = 2 * NCHK - NCHK_A
NCHK_MAX = max(NCHK_A, NCHK_B)


def _mesh():
    return plsc.VectorSubcoreMesh(core_axis_name="c", subcore_axis_name="s")


def _zero_rows(buf, d):
    def fill(r, _):
        for k in range(d // 16):
            buf[r, pl.ds(k * 16, 16)] = jnp.zeros((16,), jnp.float32)
        return 0
    lax.fori_loop(0, CH, fill, 0)


def _deg_kernel(dst2d):
    @functools.partial(
        pl.kernel,
        mesh=_mesh(),
        compiler_params=pltpu.CompilerParams(use_tc_tiling_on_sc=False),
        out_type=jax.ShapeDtypeStruct((NC, NP, 16), jnp.float32),
        scratch_types=[
            pltpu.VMEM((NCHK, CH), jnp.int32),
            pltpu.VMEM((CH, 16), jnp.float32),
            pltpu.VMEM_SHARED((NP, 16), jnp.float32),
        ],
    )
    def k(dst_hbm, out_hbm, didx, ones, acc_sh):
        c = lax.axis_index("c")
        s = lax.axis_index("s")
        wid = c * NS + s
        _zero_rows(ones, 16)
        for b in range(RB):
            pltpu.sync_copy(ones, acc_sh.at[pl.ds(s * RPT + b * CH, CH)])
        def fill(r, _):
            ones[r, pl.ds(0, 16)] = jnp.ones((16,), jnp.float32)
            return 0
        lax.fori_loop(0, CH, fill, 0)
        pltpu.sync_copy(dst_hbm.at[pl.ds(wid * NCHK, NCHK)], didx)
        plsc.subcore_barrier()
        def step(j, _):
            pltpu.sync_copy(ones, acc_sh.at[didx.at[j]], add=True)
            return 0
        lax.fori_loop(0, NCHK, step, 0)
        plsc.subcore_barrier()
        for b in range(RB):
            r0 = s * RPT + b * CH
            pltpu.sync_copy(acc_sh.at[pl.ds(r0, CH)], out_hbm.at[c, pl.ds(r0, CH)])

    return k(dst2d)


def _spmm_kernel(src2d, dst2d, g, d):
    @functools.partial(
        pl.kernel,
        mesh=_mesh(),
        compiler_params=pltpu.CompilerParams(use_tc_tiling_on_sc=False),
        out_type=jax.ShapeDtypeStruct((NC, NP, d), jnp.float32),
        scratch_types=[
            pltpu.VMEM((NCHK_MAX, CH), jnp.int32),
            pltpu.VMEM((NCHK_MAX, CH), jnp.int32),
            pltpu.VMEM((CH, d), jnp.float32),
            pltpu.VMEM((CH, d), jnp.float32),
            pltpu.VMEM_SHARED((NP, d), jnp.float32),
            pltpu.SemaphoreType.DMA,
            pltpu.SemaphoreType.DMA,
            pltpu.SemaphoreType.DMA,
            pltpu.SemaphoreType.DMA,
        ],
    )
    def k(src_hbm, dst_hbm, g_hbm, out_hbm, sidx, didx, r0, r1, acc_sh,
          g0, g1, s0, s1):
        rows = [r0, r1]
        gsem = [g0, g1]
        ssem = [s0, s1]
        c = lax.axis_index("c")
        s = lax.axis_index("s")

        def fire_gather(j, slot):
            pltpu.async_copy(g_hbm.at[sidx.at[j]], rows[slot], gsem[slot])

        def wait_gather(slot):
            pltpu.make_async_copy(g_hbm.at[sidx.at[0]], rows[slot],
                                  gsem[slot]).wait()

        def fire_scatter(j, slot):
            pltpu.async_copy(rows[slot], acc_sh.at[didx.at[j]], ssem[slot],
                             add=True)

        def wait_scatter(slot):
            pltpu.make_async_copy(rows[slot], acc_sh.at[didx.at[0]],
                                  ssem[slot]).wait()

        _zero_rows(r0, d)
        for b in range(RB):
            pltpu.sync_copy(r0, acc_sh.at[pl.ds(s * RPT + b * CH, CH)])

        def run(nchk, base):
            pltpu.sync_copy(src_hbm.at[pl.ds(base, nchk)],
                            sidx.at[pl.ds(0, nchk)])
            pltpu.sync_copy(dst_hbm.at[pl.ds(base, nchk)],
                            didx.at[pl.ds(0, nchk)])
            fire_gather(0, 0)
            wait_gather(0)
            fire_gather(1, 1)
            fire_scatter(0, 0)

            def step(i, _):
                j = 2 * i + 1
                wait_gather(1)
                wait_scatter(0)
                fire_gather(j + 1, 0)
                fire_scatter(j, 1)
                wait_gather(0)
                wait_scatter(1)
                fire_gather(j + 2, 1)
                fire_scatter(j + 1, 0)
                return 0
            lax.fori_loop(0, nchk // 2 - 1, step, 0)
            wait_gather(1)
            fire_scatter(nchk - 1, 1)
            wait_scatter(0)
            wait_scatter(1)

        @pl.when(c == 0)
        def _():
            run(NCHK_A, s * NCHK_A)

        @pl.when(c == 1)
        def _():
            run(NCHK_B, NS * NCHK_A + s * NCHK_B)
        plsc.subcore_barrier()
        for b in range(RB):
            r0 = s * RPT + b * CH
            pltpu.sync_copy(acc_sh.at[pl.ds(r0, CH)], out_hbm.at[c, pl.ds(r0, CH)])

    return k(src2d, dst2d, g)


def _prep1_body(x_ref, w_ref, deg_ref, g_ref):
    hist = deg_ref[0, :, 0:1] + deg_ref[1, :, 0:1]
    dis = lax.rsqrt(hist + 1.0)
    h = jnp.dot(x_ref[...], w_ref[...], preferred_element_type=jnp.float32)
    g_ref[...] = h * dis


def _prep2_body(acc_ref, g1_ref, deg_ref, w_ref, b1_ref, g2_ref):
    hist = deg_ref[0, :, 0:1] + deg_ref[1, :, 0:1]
    dis = lax.rsqrt(hist + 1.0)
    z = dis * (acc_ref[0] + acc_ref[1] + g1_ref[...]) + b1_ref[...]
    z = jnp.maximum(z, 0.0)
    h2 = jnp.dot(z, w_ref[...], preferred_element_type=jnp.float32)
    g2_ref[...] = h2 * dis


def _final_body(acc_ref, g2_ref, deg_ref, b2_ref, batch_ref, out_ref):
    hist = deg_ref[0, :, 0:1] + deg_ref[1, :, 0:1]
    dis = lax.rsqrt(hist + 1.0)
    out2 = dis * (acc_ref[0] + acc_ref[1] + g2_ref[...]) + b2_ref[...]
    ohT = (lax.broadcasted_iota(jnp.int32, (N_GRAPHS, NP), 0)
           == batch_ref[...]).astype(jnp.float32)
    sums = jnp.dot(ohT, out2, preferred_element_type=jnp.float32)
    counts = jnp.sum(ohT, axis=1, keepdims=True)
    pooled = sums / jnp.maximum(counts, 1.0)
    colmask = lax.broadcasted_iota(jnp.int32, (N_GRAPHS, D_OUT_PAD), 1) < D_OUT
    xm = jnp.where(colmask, pooled, -1e30)
    m = jnp.max(xm, axis=1, keepdims=True)
    e = jnp.where(colmask, jnp.exp(xm - m), 0.0)
    lse = jnp.log(jnp.sum(e, axis=1, keepdims=True)) + m
    out_ref[...] = pooled - lse


def kernel(x, edge_index, batch, W1, b1, W2, b2):
    f32 = jnp.float32
    pad = N_NODES + jnp.arange(EP - N_EDGES, dtype=jnp.int32) % (NP - N_NODES)
    src = jnp.concatenate([edge_index[0], pad])
    dst = jnp.concatenate([edge_index[1], pad])
    src2d = src.reshape(EP // CH, CH)
    dst2d = dst.reshape(EP // CH, CH)
    xp = jnp.zeros((NP, D_IN), f32).at[:N_NODES].set(x)
    W2p = jnp.zeros((D_HID, D_OUT_PAD), f32).at[:, :D_OUT].set(W2)
    b1r = b1.reshape(1, D_HID)
    b2r = jnp.zeros((1, D_OUT_PAD), f32).at[0, :D_OUT].set(b2)
    batch2d = jnp.full((1, NP), N_GRAPHS, jnp.int32).at[0, :N_NODES].set(batch)

    degp = _deg_kernel(dst2d)

    nblk, rows = 8, NP // 8
    g1 = pl.pallas_call(
        _prep1_body,
        grid=(nblk,),
        in_specs=[
            pl.BlockSpec((rows, D_IN), lambda i: (i, 0)),
            pl.BlockSpec((D_IN, D_HID), lambda i: (0, 0)),
            pl.BlockSpec((NC, rows, 16), lambda i: (0, i, 0)),
        ],
        out_specs=pl.BlockSpec((rows, D_HID), lambda i: (i, 0)),
        out_shape=jax.ShapeDtypeStruct((NP, D_HID), f32),
    )(xp, W1, degp)

    acc1 = _spmm_kernel(src2d, dst2d, g1, D_HID)

    g2 = pl.pallas_call(
        _prep2_body,
        grid=(nblk,),
        in_specs=[
            pl.BlockSpec((NC, rows, D_HID), lambda i: (0, i, 0)),
            pl.BlockSpec((rows, D_HID), lambda i: (i, 0)),
            pl.BlockSpec((NC, rows, 16), lambda i: (0, i, 0)),
            pl.BlockSpec((D_HID, D_OUT_PAD), lambda i: (0, 0)),
            pl.BlockSpec((1, D_HID), lambda i: (0, 0)),
        ],
        out_specs=pl.BlockSpec((rows, D_OUT_PAD), lambda i: (i, 0)),
        out_shape=jax.ShapeDtypeStruct((NP, D_OUT_PAD), f32),
    )(acc1, g1, degp, W2p, b1r)

    acc2 = _spmm_kernel(src2d, dst2d, g2, D_OUT_PAD)

    res = pl.pallas_call(
        _final_body,
        out_shape=jax.ShapeDtypeStruct((N_GRAPHS, D_OUT_PAD), f32),
    )(acc2, g2, degp, b2r, batch2d)
    return res[:, :D_OUT]


# speedup vs baseline: 2.7572x; 1.2984x over previous
import functools

import jax
import jax.numpy as jnp
from jax import lax
from jax.experimental import pallas as pl
from jax.experimental.pallas import tpu as pltpu
from jax.experimental.pallas import tpu_sc as plsc

N_NODES = 10000
N_EDGES = 320000
D_IN = 128
D_HID = 128
D_OUT = 2
D_OUT_PAD = 16
N_GRAPHS = 64

NC = 2
NS = 16
NW = NC * NS

NP = 10240
EP = 327680
CH = 64
NCHK = EP // (NW * CH)
RPT = NP // NS
RB = RPT // CH
NCHK_A = 160
NCHK_B = 2 * NCHK - NCHK_A
NCHK_MAX = max(NCHK_A, NCHK_B)


def _mesh():
    return plsc.VectorSubcoreMesh(core_axis_name="c", subcore_axis_name="s")


def _zero_rows(buf, d):
    def fill(r, _):
        for k in range(d // 16):
            buf[r, pl.ds(k * 16, 16)] = jnp.zeros((16,), jnp.float32)
        return 0
    lax.fori_loop(0, CH, fill, 0)


def _deg_kernel(dst2d):
    @functools.partial(
        pl.kernel,
        mesh=_mesh(),
        compiler_params=pltpu.CompilerParams(use_tc_tiling_on_sc=False),
        out_type=jax.ShapeDtypeStruct((NC, NP, 16), jnp.float32),
        scratch_types=[
            pltpu.VMEM((NCHK, CH), jnp.int32),
            pltpu.VMEM((CH, 16), jnp.float32),
            pltpu.VMEM_SHARED((NP, 16), jnp.float32),
        ],
    )
    def k(dst_hbm, out_hbm, didx, ones, acc_sh):
        c = lax.axis_index("c")
        s = lax.axis_index("s")
        wid = c * NS + s
        _zero_rows(ones, 16)
        for b in range(RB):
            pltpu.sync_copy(ones, acc_sh.at[pl.ds(s * RPT + b * CH, CH)])
        def fill(r, _):
            ones[r, pl.ds(0, 16)] = jnp.ones((16,), jnp.float32)
            return 0
        lax.fori_loop(0, CH, fill, 0)
        pltpu.sync_copy(dst_hbm.at[pl.ds(wid * NCHK, NCHK)], didx)
        plsc.subcore_barrier()
        def step(j, _):
            pltpu.sync_copy(ones, acc_sh.at[didx.at[j]], add=True)
            return 0
        lax.fori_loop(0, NCHK, step, 0)
        plsc.subcore_barrier()
        for b in range(RB):
            r0 = s * RPT + b * CH
            pltpu.sync_copy(acc_sh.at[pl.ds(r0, CH)], out_hbm.at[c, pl.ds(r0, CH)])

    return k(dst2d)


def _spmm_kernel(src2d, dst2d, g, d):
    @functools.partial(
        pl.kernel,
        mesh=_mesh(),
        compiler_params=pltpu.CompilerParams(use_tc_tiling_on_sc=False),
        out_type=jax.ShapeDtypeStruct((NC, NP, d), jnp.float32),
        scratch_types=[
            pltpu.VMEM((NCHK_MAX, CH), jnp.int32),
            pltpu.VMEM((NCHK_MAX, CH), jnp.int32),
            pltpu.VMEM((CH, d), jnp.float32),
            pltpu.VMEM((CH, d), jnp.float32),
            pltpu.VMEM_SHARED((NP, d), jnp.float32),
            pltpu.SemaphoreType.DMA,
            pltpu.SemaphoreType.DMA,
            pltpu.SemaphoreType.DMA,
            pltpu.SemaphoreType.DMA,
        ],
    )
    def k(src_hbm, dst_hbm, g_hbm, out_hbm, sidx, didx, r0, r1, acc_sh,
          g0, g1, s0, s1):
        rows = [r0, r1]
        gsem = [g0, g1]
        ssem = [s0, s1]
        c = lax.axis_index("c")
        s = lax.axis_index("s")

        def fire_gather(j, slot):
            pltpu.async_copy(g_hbm.at[sidx.at[j]], rows[slot], gsem[slot])

        def wait_gather(slot):
            pltpu.make_async_copy(g_hbm.at[sidx.at[0]], rows[slot],
                                  gsem[slot]).wait()

        def fire_scatter(j, slot):
            pltpu.async_copy(rows[slot], acc_sh.at[didx.at[j]], ssem[slot],
                             add=True)

        def wait_scatter(slot):
            pltpu.make_async_copy(rows[slot], acc_sh.at[didx.at[0]],
                                  ssem[slot]).wait()

        _zero_rows(r0, d)
        for b in range(RB):
            pltpu.sync_copy(r0, acc_sh.at[pl.ds(s * RPT + b * CH, CH)])

        def run(nchk, base):
            pltpu.sync_copy(src_hbm.at[pl.ds(base, nchk)],
                            sidx.at[pl.ds(0, nchk)])
            pltpu.sync_copy(dst_hbm.at[pl.ds(base, nchk)],
                            didx.at[pl.ds(0, nchk)])
            fire_gather(0, 0)
            wait_gather(0)
            fire_gather(1, 1)
            fire_scatter(0, 0)

            def step(i, _):
                j = 2 * i + 1
                wait_gather(1)
                wait_scatter(0)
                fire_gather(j + 1, 0)
                fire_scatter(j, 1)
                wait_gather(0)
                wait_scatter(1)
                fire_gather(j + 2, 1)
                fire_scatter(j + 1, 0)
                return 0
            lax.fori_loop(0, nchk // 2 - 1, step, 0)
            wait_gather(1)
            fire_scatter(nchk - 1, 1)
            wait_scatter(0)
            wait_scatter(1)

        run(NCHK, (c * NS + s) * NCHK)
        plsc.subcore_barrier()
        for b in range(RB):
            r0 = s * RPT + b * CH
            pltpu.sync_copy(acc_sh.at[pl.ds(r0, CH)], out_hbm.at[c, pl.ds(r0, CH)])

    return k(src2d, dst2d, g)


CH2 = 128
NCHK2 = EP // (NW * CH2)


def _spmm16_kernel(src2d, dst2d, g):
    d = D_OUT_PAD

    @functools.partial(
        pl.kernel,
        mesh=_mesh(),
        compiler_params=pltpu.CompilerParams(use_tc_tiling_on_sc=False),
        out_type=jax.ShapeDtypeStruct((NC, NP, d), jnp.float32),
        scratch_types=[
            pltpu.VMEM((NCHK2, CH2), jnp.int32),
            pltpu.VMEM((NCHK2, CH2), jnp.int32),
            pltpu.VMEM((CH2, d), jnp.float32),
            pltpu.VMEM((CH2, d), jnp.float32),
            pltpu.VMEM((CH2, d), jnp.float32),
            pltpu.VMEM((CH2, d), jnp.float32),
            pltpu.VMEM_SHARED((NP, d), jnp.float32),
            pltpu.VMEM_SHARED((NP, d), jnp.float32),
            pltpu.SemaphoreType.DMA,
            pltpu.SemaphoreType.DMA,
            pltpu.SemaphoreType.DMA,
            pltpu.SemaphoreType.DMA,
            pltpu.SemaphoreType.DMA,
            pltpu.SemaphoreType.DMA,
            pltpu.SemaphoreType.DMA,
            pltpu.SemaphoreType.DMA,
        ],
    )
    def k(src_hbm, dst_hbm, g_hbm, out_hbm, sidx, didx, r0, r1, r2, r3,
          gst, acc_sh, g0, g1, g2, g3, s0, s1, s2, s3):
        rows = [r0, r1, r2, r3]
        gsem = [g0, g1, g2, g3]
        ssem = [s0, s1, s2, s3]
        c = lax.axis_index("c")
        s = lax.axis_index("s")
        wid = c * NS + s

        def fire_gather(j, slot):
            pltpu.async_copy(gst.at[sidx.at[j]], rows[slot], gsem[slot])

        def wait_gather(slot):
            pltpu.make_async_copy(gst.at[sidx.at[0]], rows[slot],
                                  gsem[slot]).wait()

        def fire_scatter(j, slot):
            pltpu.async_copy(rows[slot], acc_sh.at[didx.at[j]], ssem[slot],
                             add=True)

        def wait_scatter(slot):
            pltpu.make_async_copy(rows[slot], acc_sh.at[didx.at[0]],
                                  ssem[slot]).wait()

        _zero_rows(r0, d)
        for b in range(RPT // CH2):
            pltpu.sync_copy(r0, acc_sh.at[pl.ds(s * RPT + b * CH2, CH2)])
        pltpu.sync_copy(g_hbm.at[pl.ds(s * RPT, RPT)], gst.at[pl.ds(s * RPT, RPT)])
        pltpu.sync_copy(src_hbm.at[pl.ds(wid * NCHK2, NCHK2)], sidx)
        pltpu.sync_copy(dst_hbm.at[pl.ds(wid * NCHK2, NCHK2)], didx)
        plsc.subcore_barrier()
        fire_gather(0, 0)
        fire_gather(1, 1)
        wait_gather(0)
        fire_gather(2, 2)
        fire_scatter(0, 0)
        wait_gather(1)
        fire_gather(3, 3)
        fire_scatter(1, 1)

        def step(i, _):
            j0 = 4 * i + 2
            for kk in range(4):
                slot = (2 + kk) % 4
                nslot = slot ^ 2
                jj = j0 + kk
                wait_gather(slot)
                wait_scatter(nslot)
                fire_gather(jj + 2, nslot)
                fire_scatter(jj, slot)
            return 0
        lax.fori_loop(0, (NCHK2 - 4) // 4, step, 0)
        wait_gather((NCHK2 - 2) % 4)
        wait_scatter(NCHK2 % 4)
        fire_scatter(NCHK2 - 2, (NCHK2 - 2) % 4)
        wait_gather((NCHK2 - 1) % 4)
        wait_scatter((NCHK2 + 1) % 4)
        fire_scatter(NCHK2 - 1, (NCHK2 - 1) % 4)
        wait_scatter((NCHK2 - 2) % 4)
        wait_scatter((NCHK2 - 1) % 4)
        plsc.subcore_barrier()
        for b in range(RPT // CH2):
            r0b = s * RPT + b * CH2
            pltpu.sync_copy(acc_sh.at[pl.ds(r0b, CH2)],
                            out_hbm.at[c, pl.ds(r0b, CH2)])

    return k(src2d, dst2d, g)


def _prep1_body(x_ref, w_ref, deg_ref, g_ref):
    hist = deg_ref[0, :, 0:1] + deg_ref[1, :, 0:1]
    dis = lax.rsqrt(hist + 1.0)
    h = jnp.dot(x_ref[...], w_ref[...], preferred_element_type=jnp.float32)
    g_ref[...] = h * dis


def _prep2_body(acc_ref, g1_ref, deg_ref, w_ref, b1_ref, g2_ref):
    hist = deg_ref[0, :, 0:1] + deg_ref[1, :, 0:1]
    dis = lax.rsqrt(hist + 1.0)
    z = dis * (acc_ref[0] + acc_ref[1] + g1_ref[...]) + b1_ref[...]
    z = jnp.maximum(z, 0.0)
    h2 = jnp.dot(z, w_ref[...], preferred_element_type=jnp.float32)
    g2_ref[...] = h2 * dis


def _final_body(acc_ref, g2_ref, deg_ref, b2_ref, batch_ref, out_ref):
    hist = deg_ref[0, :, 0:1] + deg_ref[1, :, 0:1]
    dis = lax.rsqrt(hist + 1.0)
    out2 = dis * (acc_ref[0] + acc_ref[1] + g2_ref[...]) + b2_ref[...]
    ohT = (lax.broadcasted_iota(jnp.int32, (N_GRAPHS, NP), 0)
           == batch_ref[...]).astype(jnp.float32)
    sums = jnp.dot(ohT, out2, preferred_element_type=jnp.float32)
    counts = jnp.sum(ohT, axis=1, keepdims=True)
    pooled = sums / jnp.maximum(counts, 1.0)
    colmask = lax.broadcasted_iota(jnp.int32, (N_GRAPHS, D_OUT_PAD), 1) < D_OUT
    xm = jnp.where(colmask, pooled, -1e30)
    m = jnp.max(xm, axis=1, keepdims=True)
    e = jnp.where(colmask, jnp.exp(xm - m), 0.0)
    lse = jnp.log(jnp.sum(e, axis=1, keepdims=True)) + m
    out_ref[...] = pooled - lse


def kernel(x, edge_index, batch, W1, b1, W2, b2):
    f32 = jnp.float32
    pad = N_NODES + jnp.arange(EP - N_EDGES, dtype=jnp.int32) % (NP - N_NODES)
    src = jnp.concatenate([edge_index[0], pad])
    dst = jnp.concatenate([edge_index[1], pad])
    src2d = src.reshape(EP // CH, CH)
    dst2d = dst.reshape(EP // CH, CH)
    src2db = src.reshape(EP // CH2, CH2)
    dst2db = dst.reshape(EP // CH2, CH2)
    xp = jnp.zeros((NP, D_IN), f32).at[:N_NODES].set(x)
    W2p = jnp.zeros((D_HID, D_OUT_PAD), f32).at[:, :D_OUT].set(W2)
    b1r = b1.reshape(1, D_HID)
    b2r = jnp.zeros((1, D_OUT_PAD), f32).at[0, :D_OUT].set(b2)
    batch2d = jnp.full((1, NP), N_GRAPHS, jnp.int32).at[0, :N_NODES].set(batch)

    degp = _deg_kernel(dst2d)

    nblk, rows = 8, NP // 8
    g1 = pl.pallas_call(
        _prep1_body,
        grid=(nblk,),
        in_specs=[
            pl.BlockSpec((rows, D_IN), lambda i: (i, 0)),
            pl.BlockSpec((D_IN, D_HID), lambda i: (0, 0)),
            pl.BlockSpec((NC, rows, 16), lambda i: (0, i, 0)),
        ],
        out_specs=pl.BlockSpec((rows, D_HID), lambda i: (i, 0)),
        out_shape=jax.ShapeDtypeStruct((NP, D_HID), f32),
    )(xp, W1, degp)

    acc1 = _spmm_kernel(src2d, dst2d, g1, D_HID)

    g2 = pl.pallas_call(
        _prep2_body,
        grid=(nblk,),
        in_specs=[
            pl.BlockSpec((NC, rows, D_HID), lambda i: (0, i, 0)),
            pl.BlockSpec((rows, D_HID), lambda i: (i, 0)),
            pl.BlockSpec((NC, rows, 16), lambda i: (0, i, 0)),
            pl.BlockSpec((D_HID, D_OUT_PAD), lambda i: (0, 0)),
            pl.BlockSpec((1, D_HID), lambda i: (0, 0)),
        ],
        out_specs=pl.BlockSpec((rows, D_OUT_PAD), lambda i: (i, 0)),
        out_shape=jax.ShapeDtypeStruct((NP, D_OUT_PAD), f32),
    )(acc1, g1, degp, W2p, b1r)

    acc2 = _spmm16_kernel(src2db, dst2db, g2)

    res = pl.pallas_call(
        _final_body,
        out_shape=jax.ShapeDtypeStruct((N_GRAPHS, D_OUT_PAD), f32),
    )(acc2, g2, degp, b2r, batch2d)
    return res[:, :D_OUT]
